# Initial kernel scaffold; baseline (speedup 1.0000x reference)
import jax
import jax.numpy as jnp
from jax.experimental import pallas as pl


def kernel(input_ids, token_type_ids, W_word, pos_table, type_table, gamma, beta):
    raise NotImplementedError("write your pallas kernel here")


# speedup vs baseline: 3.1571x; 3.1571x over previous
import functools

import jax
import jax.numpy as jnp
from jax.experimental import pallas as pl
from jax.experimental.pallas import tpu as pltpu


def _fused_kernel(inp_ref, tt_ref, w_ref, pos_ref, type_ref, gamma_ref,
                  beta_ref, out_ref):
    x = inp_ref[0]
    acc = jnp.dot(x, w_ref[...], preferred_element_type=jnp.float32)
    tt = tt_ref[0, 0, :]
    t0 = type_ref[0, :]
    dt = type_ref[1, :] - type_ref[0, :]
    emb = (acc + pos_ref[...] + t0[None, :]
           + tt.astype(jnp.float32)[:, None] * dt[None, :])
    mu = jnp.mean(emb, axis=1, keepdims=True)
    d = emb - mu
    var = jnp.mean(d * d, axis=1, keepdims=True)
    out_ref[0] = (d * jax.lax.rsqrt(var + 1e-12)) * gamma_ref[...] + beta_ref[...]


@functools.partial(jax.jit, static_argnames=())
def kernel(input_ids, token_type_ids, W_word, pos_table, type_table, gamma, beta):
    B, S, V = input_ids.shape
    H = W_word.shape[1]
    tt3 = token_type_ids.reshape(B, 1, S)
    gamma2 = gamma.reshape(1, H)
    beta2 = beta.reshape(1, H)

    grid = (B,)
    out = pl.pallas_call(
        _fused_kernel,
        grid=grid,
        in_specs=[
            pl.BlockSpec((1, S, V), lambda b: (b, 0, 0)),
            pl.BlockSpec((1, 1, S), lambda b: (b, 0, 0)),
            pl.BlockSpec((V, H), lambda b: (0, 0)),
            pl.BlockSpec((S, H), lambda b: (0, 0)),
            pl.BlockSpec((2, H), lambda b: (0, 0)),
            pl.BlockSpec((1, H), lambda b: (0, 0)),
            pl.BlockSpec((1, H), lambda b: (0, 0)),
        ],
        out_specs=pl.BlockSpec((1, S, H), lambda b: (b, 0, 0)),
        out_shape=jax.ShapeDtypeStruct((B, S, H), jnp.float32),
        compiler_params=pltpu.CompilerParams(
            dimension_semantics=("parallel",),
        ),
    )(input_ids, tt3, W_word, pos_table, type_table, gamma2, beta2)
    return out
